# initial kernel scaffold (unmeasured)
import jax
import jax.numpy as jnp
from jax import lax
from jax.experimental import pallas as pl
from jax.experimental.pallas import tpu as pltpu


def kernel(
    x,
):
    def body(*refs):
        pass

    out_shape = jax.ShapeDtypeStruct(..., jnp.float32)
    return pl.pallas_call(body, out_shape=out_shape)(...)


# baseline (device time: 64926 ns/iter reference)
import jax
import jax.numpy as jnp
from jax import lax
from jax.experimental import pallas as pl
from jax.experimental.pallas import tpu as pltpu

M, N = 512, 512
N_STAGES = 5


def kernel(x):
    x2 = x.reshape(M, N)

    def body(x_ref, out_ref, acc_ref, send_buf, recv_bufs, send_sems, recv_sems):
        my_x = lax.axis_index("x")
        my_y = lax.axis_index("y")
        my_z = lax.axis_index("z")

        partners = [
            (1 - my_x, my_y, my_z),
            (my_x, my_y ^ 1, my_z),
            (my_x, my_y ^ 2, my_z),
            (my_x, my_y, my_z ^ 1),
            (my_x, my_y, my_z ^ 2),
        ]

        acc_ref[...] = x_ref[...]
        for s in range(N_STAGES):
            send_buf[...] = acc_ref[...].astype(jnp.bfloat16)
            rdma = pltpu.make_async_remote_copy(
                src_ref=send_buf,
                dst_ref=recv_bufs.at[s],
                send_sem=send_sems.at[s],
                recv_sem=recv_sems.at[s],
                device_id=partners[s],
                device_id_type=pl.DeviceIdType.MESH,
            )
            rdma.start()
            rdma.wait()
            acc_ref[...] += recv_bufs[s].astype(jnp.float32)
        out_ref[...] = acc_ref[...]

    return pl.pallas_call(
        body,
        out_shape=jax.ShapeDtypeStruct((M, N), jnp.float32),
        in_specs=[pl.BlockSpec(memory_space=pltpu.VMEM)],
        out_specs=pl.BlockSpec(memory_space=pltpu.VMEM),
        scratch_shapes=[
            pltpu.VMEM((M, N), jnp.float32),
            pltpu.VMEM((M, N), jnp.bfloat16),
            pltpu.VMEM((N_STAGES, M, N), jnp.bfloat16),
            pltpu.SemaphoreType.DMA((N_STAGES,)),
            pltpu.SemaphoreType.DMA((N_STAGES,)),
        ],
    )(x2)


# device time: 40928 ns/iter; 1.5863x vs baseline; 1.5863x over previous
import jax
import jax.numpy as jnp
from jax import lax
from jax.experimental import pallas as pl
from jax.experimental.pallas import tpu as pltpu

M, N = 512, 512
N_STAGES = 5
C = 4
ROWS = M // C


def kernel(x):
    x2 = x.reshape(M, N)

    def body(x_ref, out_ref, acc_ref, send_bufs, recv_bufs, send_sems, recv_sems):
        my_x = lax.axis_index("x")
        my_y = lax.axis_index("y")
        my_z = lax.axis_index("z")

        partners = [
            (1 - my_x, my_y, my_z),
            (my_x, my_y ^ 1, my_z),
            (my_x, my_y ^ 2, my_z),
            (my_x, my_y, my_z ^ 1),
            (my_x, my_y, my_z ^ 2),
        ]

        barrier_sem = pltpu.get_barrier_semaphore()
        for p in partners:
            pl.semaphore_signal(
                barrier_sem, inc=1, device_id=p,
                device_id_type=pl.DeviceIdType.MESH,
            )
        pl.semaphore_wait(barrier_sem, N_STAGES)

        acc_ref[...] = x_ref[...]

        rdmas = {}
        for s in range(N_STAGES):
            for c in range(C):
                rows = pl.ds(c * ROWS, ROWS)
                if s > 0:
                    rdmas[(s - 1, c)].wait_recv()
                    acc_ref[rows, :] += recv_bufs[s - 1, rows, :].astype(
                        jnp.float32
                    )
                send_bufs[s, rows, :] = acc_ref[rows, :].astype(jnp.bfloat16)
                r = pltpu.make_async_remote_copy(
                    src_ref=send_bufs.at[s, rows, :],
                    dst_ref=recv_bufs.at[s, rows, :],
                    send_sem=send_sems.at[s, c],
                    recv_sem=recv_sems.at[s, c],
                    device_id=partners[s],
                    device_id_type=pl.DeviceIdType.MESH,
                )
                r.start()
                rdmas[(s, c)] = r

        s = N_STAGES - 1
        for c in range(C):
            rows = pl.ds(c * ROWS, ROWS)
            rdmas[(s, c)].wait_recv()
            acc_ref[rows, :] += recv_bufs[s, rows, :].astype(jnp.float32)
        out_ref[...] = acc_ref[...]

        for s in range(N_STAGES):
            for c in range(C):
                rdmas[(s, c)].wait_send()

    return pl.pallas_call(
        body,
        out_shape=jax.ShapeDtypeStruct((M, N), jnp.float32),
        in_specs=[pl.BlockSpec(memory_space=pltpu.VMEM)],
        out_specs=pl.BlockSpec(memory_space=pltpu.VMEM),
        scratch_shapes=[
            pltpu.VMEM((M, N), jnp.float32),
            pltpu.VMEM((N_STAGES, M, N), jnp.bfloat16),
            pltpu.VMEM((N_STAGES, M, N), jnp.bfloat16),
            pltpu.SemaphoreType.DMA((N_STAGES, C)),
            pltpu.SemaphoreType.DMA((N_STAGES, C)),
        ],
        compiler_params=pltpu.CompilerParams(collective_id=0),
    )(x2)
